# baseline (speedup 1.0000x reference)
import functools

import jax
import jax.numpy as jnp
from jax.experimental import pallas as pl
from jax.experimental.pallas import tpu as pltpu

S, H, I, E = 2048, 1024, 2816, 8
IT = 256
N_IT = I // IT


def _moe_kernel(xt_ref, rw_w_ref, rb_ref, g_ref, u_ref, d_ref,
                out_ref, loss_ref, rwst_ref, acc_ref):
    e = pl.program_id(0)
    it = pl.program_id(1)

    @pl.when(jnp.logical_and(e == 0, it == 0))
    def _router():
        logits_t = jax.lax.dot_general(
            rw_w_ref[...].astype(jnp.bfloat16), xt_ref[...],
            (((1,), (0,)), ((), ())),
            preferred_element_type=jnp.float32) + rb_ref[...].reshape(E, 1)
        m = jnp.max(logits_t, axis=0, keepdims=True)
        ex = jnp.exp(logits_t - m)
        rw_t = ex / jnp.sum(ex, axis=0, keepdims=True)
        rwst_ref[...] = rw_t
        diff = rw_t - (1.0 / E)
        loss_ref[...] = (jnp.mean(diff * diff) * 0.01).reshape(1, 1)

    xt = xt_ref[...]
    g = g_ref[0].astype(jnp.bfloat16)
    u = u_ref[0].astype(jnp.bfloat16)
    dwn = d_ref[0].astype(jnp.bfloat16)
    gate_t = jax.lax.dot_general(g, xt, (((1,), (0,)), ((), ())),
                                 preferred_element_type=jnp.float32)
    up_t = jax.lax.dot_general(u, xt, (((1,), (0,)), ((), ())),
                               preferred_element_type=jnp.float32)
    t_t = gate_t * jax.nn.sigmoid(gate_t) * up_t
    w_e = rwst_ref[pl.ds(e, 1), :]
    t16 = (t_t * w_e).astype(jnp.bfloat16)
    contrib = jax.lax.dot_general(dwn, t16, (((1,), (0,)), ((), ())),
                                  preferred_element_type=jnp.float32)

    @pl.when(jnp.logical_and(e == 0, it == 0))
    def _first():
        acc_ref[...] = contrib

    @pl.when(jnp.logical_or(e > 0, it > 0))
    def _rest():
        acc_ref[...] += contrib

    @pl.when(jnp.logical_and(e == E - 1, it == N_IT - 1))
    def _emit():
        out_ref[...] = acc_ref[...].T


@functools.partial(jax.jit, static_argnames=())
def kernel(hidden_states, router_w, router_b, gate_w, up_w, down_w):
    xt = hidden_states.reshape(S, H).T.astype(jnp.bfloat16)
    rb = router_b.reshape(1, E)
    out, loss = pl.pallas_call(
        _moe_kernel,
        grid=(E, N_IT),
        in_specs=[
            pl.BlockSpec((H, S), lambda e, i: (0, 0)),
            pl.BlockSpec((E, H), lambda e, i: (0, 0)),
            pl.BlockSpec((1, E), lambda e, i: (0, 0)),
            pl.BlockSpec((1, IT, H), lambda e, i: (e, i, 0)),
            pl.BlockSpec((1, IT, H), lambda e, i: (e, i, 0)),
            pl.BlockSpec((1, H, IT), lambda e, i: (e, 0, i)),
        ],
        out_specs=[
            pl.BlockSpec((S, H), lambda e, i: (0, 0)),
            pl.BlockSpec((1, 1), lambda e, i: (0, 0)),
        ],
        out_shape=[
            jax.ShapeDtypeStruct((S, H), jnp.float32),
            jax.ShapeDtypeStruct((1, 1), jnp.float32),
        ],
        scratch_shapes=[pltpu.VMEM((E, S), jnp.float32),
                        pltpu.VMEM((H, S), jnp.float32)],
    )(xt, router_w, rb, gate_w, up_w, down_w)
    return out.reshape(hidden_states.shape), loss[0, 0]


# speedup vs baseline: 1.2447x; 1.2447x over previous
import functools

import jax
import jax.numpy as jnp
from jax.experimental import pallas as pl
from jax.experimental.pallas import tpu as pltpu

S, H, I, E = 2048, 1024, 2816, 8
IT = 256
N_IT = I // IT
EP = E // 2


def _moe_kernel(x_ref, rw_w_ref, rb_ref, g_ref, u_ref, d_ref,
                out_ref, loss_ref, rws_ref):
    p = pl.program_id(0)
    it = pl.program_id(1)

    @pl.when(jnp.logical_and(p == 0, it == 0))
    def _router():
        x = x_ref[...]
        logits = jax.lax.dot_general(
            x, rw_w_ref[...], (((1,), (1,)), ((), ())),
            preferred_element_type=jnp.float32) + rb_ref[0, :]
        m = jnp.max(logits, axis=-1, keepdims=True)
        ex = jnp.exp(logits - m)
        rw = ex / jnp.sum(ex, axis=-1, keepdims=True)
        rws_ref[...] = rw
        diff = rw - (1.0 / E)
        loss_ref[...] = (jnp.mean(diff * diff) * 0.01).reshape(1, 1)
        out_ref[...] = jnp.zeros(out_ref.shape, out_ref.dtype)

    x = x_ref[...].astype(jnp.bfloat16)
    lane = jax.lax.broadcasted_iota(jnp.int32, (S, E), 1)
    rw_all = rws_ref[...]
    halves = []
    for k in range(2):
        g = g_ref[k].astype(jnp.bfloat16)
        u = u_ref[k].astype(jnp.bfloat16)
        gate = jax.lax.dot_general(x, g, (((1,), (1,)), ((), ())),
                                   preferred_element_type=jnp.float32)
        up = jax.lax.dot_general(x, u, (((1,), (1,)), ((), ())),
                                 preferred_element_type=jnp.float32)
        t = gate * jax.nn.sigmoid(gate) * up
        w_e = jnp.sum(jnp.where(lane == 2 * p + k, rw_all, 0.0), axis=1,
                      keepdims=True)
        halves.append((t * w_e).astype(jnp.bfloat16))
    t_cat = jnp.concatenate(halves, axis=1)
    d_cat = jnp.concatenate([d_ref[0].astype(jnp.bfloat16),
                             d_ref[1].astype(jnp.bfloat16)], axis=1)
    out_ref[...] += jax.lax.dot_general(t_cat, d_cat,
                                        (((1,), (1,)), ((), ())),
                                        preferred_element_type=jnp.float32)


@functools.partial(jax.jit, static_argnames=())
def kernel(hidden_states, router_w, router_b, gate_w, up_w, down_w):
    x = hidden_states.reshape(S, H)
    rb = router_b.reshape(1, E)
    out, loss = pl.pallas_call(
        _moe_kernel,
        grid=(EP, N_IT),
        in_specs=[
            pl.BlockSpec((S, H), lambda p, i: (0, 0)),
            pl.BlockSpec((E, H), lambda p, i: (0, 0)),
            pl.BlockSpec((1, E), lambda p, i: (0, 0)),
            pl.BlockSpec((2, IT, H), lambda p, i: (p, i, 0)),
            pl.BlockSpec((2, IT, H), lambda p, i: (p, i, 0)),
            pl.BlockSpec((2, H, IT), lambda p, i: (p, 0, i)),
        ],
        out_specs=[
            pl.BlockSpec((S, H), lambda p, i: (0, 0)),
            pl.BlockSpec((1, 1), lambda p, i: (0, 0)),
        ],
        out_shape=[
            jax.ShapeDtypeStruct((S, H), jnp.float32),
            jax.ShapeDtypeStruct((1, 1), jnp.float32),
        ],
        scratch_shapes=[pltpu.VMEM((S, E), jnp.float32)],
    )(x, router_w, rb, gate_w, up_w, down_w)
    return out.reshape(hidden_states.shape), loss[0, 0]


# speedup vs baseline: 1.2668x; 1.0177x over previous
import functools

import jax
import jax.numpy as jnp
from jax.experimental import pallas as pl
from jax.experimental.pallas import tpu as pltpu

S, H, I, E = 2048, 1024, 2816, 8
IT = 256
N_IT = I // IT
EP = E // 4


def _moe_kernel(x_ref, rw_w_ref, rb_ref, g_ref, u_ref, d_ref,
                out_ref, loss_ref, rws_ref):
    p = pl.program_id(0)
    it = pl.program_id(1)

    @pl.when(jnp.logical_and(p == 0, it == 0))
    def _router():
        x = x_ref[...]
        logits = jax.lax.dot_general(
            x, rw_w_ref[...], (((1,), (1,)), ((), ())),
            preferred_element_type=jnp.float32) + rb_ref[0, :]
        m = jnp.max(logits, axis=-1, keepdims=True)
        ex = jnp.exp(logits - m)
        rw = ex / jnp.sum(ex, axis=-1, keepdims=True)
        rws_ref[...] = rw
        diff = rw - (1.0 / E)
        loss_ref[...] = (jnp.mean(diff * diff) * 0.01).reshape(1, 1)
        out_ref[...] = jnp.zeros(out_ref.shape, out_ref.dtype)

    x = x_ref[...].astype(jnp.bfloat16)
    lane = jax.lax.broadcasted_iota(jnp.int32, (S, E), 1)
    rw_all = rws_ref[...]
    halves = []
    for k in range(4):
        g = g_ref[k].astype(jnp.bfloat16)
        u = u_ref[k].astype(jnp.bfloat16)
        gate = jax.lax.dot_general(x, g, (((1,), (1,)), ((), ())),
                                   preferred_element_type=jnp.float32)
        up = jax.lax.dot_general(x, u, (((1,), (1,)), ((), ())),
                                 preferred_element_type=jnp.float32)
        t = gate * jax.nn.sigmoid(gate) * up
        w_e = jnp.sum(jnp.where(lane == 4 * p + k, rw_all, 0.0), axis=1,
                      keepdims=True)
        halves.append((t * w_e).astype(jnp.bfloat16))
    t_cat = jnp.concatenate(halves, axis=1)
    d_cat = jnp.concatenate([d_ref[k].astype(jnp.bfloat16)
                             for k in range(4)], axis=1)
    out_ref[...] += jax.lax.dot_general(t_cat, d_cat,
                                        (((1,), (1,)), ((), ())),
                                        preferred_element_type=jnp.float32)


@functools.partial(jax.jit, static_argnames=())
def kernel(hidden_states, router_w, router_b, gate_w, up_w, down_w):
    x = hidden_states.reshape(S, H)
    rb = router_b.reshape(1, E)
    out, loss = pl.pallas_call(
        _moe_kernel,
        grid=(EP, N_IT),
        in_specs=[
            pl.BlockSpec((S, H), lambda p, i: (0, 0)),
            pl.BlockSpec((E, H), lambda p, i: (0, 0)),
            pl.BlockSpec((1, E), lambda p, i: (0, 0)),
            pl.BlockSpec((4, IT, H), lambda p, i: (p, i, 0)),
            pl.BlockSpec((4, IT, H), lambda p, i: (p, i, 0)),
            pl.BlockSpec((4, H, IT), lambda p, i: (p, 0, i)),
        ],
        out_specs=[
            pl.BlockSpec((S, H), lambda p, i: (0, 0)),
            pl.BlockSpec((1, 1), lambda p, i: (0, 0)),
        ],
        out_shape=[
            jax.ShapeDtypeStruct((S, H), jnp.float32),
            jax.ShapeDtypeStruct((1, 1), jnp.float32),
        ],
        scratch_shapes=[pltpu.VMEM((S, E), jnp.float32)],
        compiler_params=pltpu.CompilerParams(
            vmem_limit_bytes=100 * 1024 * 1024),
    )(x, router_w, rb, gate_w, up_w, down_w)
    return out.reshape(hidden_states.shape), loss[0, 0]
